# baseline (speedup 1.0000x reference)
import jax
import jax.numpy as jnp
from jax import lax
from jax.experimental import pallas as pl
from jax.experimental.pallas import tpu as pltpu
from jax.experimental.pallas import tpu_sc as plsc

_NC = 2
_NS = 16
_NW = _NC * _NS
_L = 16
_C = 256
_NB = 2


def _gather_body(idx_hbm, table_hbm, out_hbm, idx_v, rows_v, trans_v,
                 sem_i, sem_g, sem_o):
    n_total = idx_hbm.shape[0]
    dim = out_hbm.shape[1]
    batch = out_hbm.shape[2]
    n_per_w = n_total // _NW
    n_chunks = n_per_w // _C
    wid = lax.axis_index("s") * _NC + lax.axis_index("c")
    base = wid * n_per_w
    lane = lax.iota(jnp.int32, _L)

    for b in range(_NB):
        pltpu.async_copy(
            idx_hbm.at[pl.ds(base + b * _C, _C)], idx_v.at[b], sem_i.at[b])

    @pl.loop(0, n_chunks, step=_NB)
    def _(j0):
        for b in range(_NB):
            pltpu.make_async_copy(
                idx_hbm.at[pl.ds(base, _C)], idx_v.at[b], sem_i.at[b]).wait()

            @pl.loop(0, _C, step=_L)
            def _(r):
                vals = idx_v[b, pl.ds(r, _L)]
                pltpu.async_copy(
                    table_hbm.at[vals], rows_v.at[b, pl.ds(r, _L)],
                    sem_g.at[b])
        for b in range(_NB):
            pltpu.make_async_copy(
                table_hbm.at[pl.ds(0, _C)], rows_v.at[b], sem_g.at[b]).wait()

            @pl.loop(0, dim)
            def _(i):
                col = lane * 0 + i

                @pl.loop(0, _C, step=_L)
                def _(r):
                    seg = plsc.load_gather(rows_v.at[b], [lane + r, col])
                    trans_v[b, i, pl.ds(r, _L)] = seg

            r0 = base + (j0 + b) * _C
            s = r0 // batch
            b0 = lax.rem(r0, batch)
            pltpu.async_copy(
                trans_v.at[b],
                out_hbm.at[s, pl.ds(0, dim), pl.ds(b0, _C)], sem_o.at[b])
            off_next = base + lax.rem(j0 + _NB + b, n_chunks) * _C
            pltpu.async_copy(
                idx_hbm.at[pl.ds(off_next, _C)], idx_v.at[b], sem_i.at[b])
        for b in range(_NB):
            pltpu.make_async_copy(
                out_hbm.at[0, pl.ds(0, dim), pl.ds(0, _C)],
                trans_v.at[b], sem_o.at[b]).wait()

    for b in range(_NB):
        pltpu.make_async_copy(
            idx_hbm.at[pl.ds(base, _C)], idx_v.at[b], sem_i.at[b]).wait()


def kernel(x, src_weight):
    batch, seq = x.shape
    _, dim = src_weight.shape
    n_total = batch * seq
    idx = x.T.reshape(n_total).astype(jnp.int32)

    mesh = plsc.VectorSubcoreMesh(core_axis_name="c", subcore_axis_name="s")
    out_p = pl.kernel(
        _gather_body,
        out_type=jax.ShapeDtypeStruct((seq, dim, batch), jnp.float32),
        mesh=mesh,
        scratch_types=[
            pltpu.VMEM((_NB, _C), jnp.int32),
            pltpu.VMEM((_NB, _C, dim), jnp.float32),
            pltpu.VMEM((_NB, dim, _C), jnp.float32),
            pltpu.SemaphoreType.DMA((_NB,)),
            pltpu.SemaphoreType.DMA((_NB,)),
            pltpu.SemaphoreType.DMA((_NB,)),
        ],
        compiler_params=pltpu.CompilerParams(
            use_tc_tiling_on_sc=False, needs_layout_passes=False),
    )(idx, src_weight)
    return out_p.transpose(2, 0, 1)
